# baseline (speedup 1.0000x reference)
import functools

import jax
import jax.numpy as jnp
from jax import lax
from jax.experimental import pallas as pl
from jax.experimental.pallas import tpu as pltpu
from jax.experimental.pallas import tpu_sc as plsc

D = 64
EPS = 1e-5
L = 16
NC = 2
NS = 16
NW = NC * NS
CH = 128


def _rsqrt_vec(x):
    i = lax.bitcast_convert_type(x, jnp.int32)
    i = jnp.int32(0x5F375A86) - lax.shift_right_logical(i, 1)
    y = lax.bitcast_convert_type(i, jnp.float32)
    for _ in range(3):
        y = y * (jnp.float32(1.5) - jnp.float32(0.5) * x * y * y)
    return y


def _butterfly_perms():
    lane = lax.iota(jnp.int32, L)
    return [lax.bitwise_xor(lane, jnp.int32(step)) for step in (1, 2, 4, 8)]


def _take16(v, p):
    dnums = lax.GatherDimensionNumbers(
        offset_dims=(), collapsed_slice_dims=(0,), start_index_map=(0,)
    )
    return lax.gather(
        v,
        p.reshape(L, 1),
        dimension_numbers=dnums,
        slice_sizes=(1,),
        mode=lax.GatherScatterMode.PROMISE_IN_BOUNDS,
    )


def _lane_sum(v, perms):
    for p in perms:
        v = v + _take16(v, p)
    return v


def _make_kernel(n_rows):
    rows_per_worker = n_rows // NW
    n_chunks = rows_per_worker // CH
    mesh = plsc.VectorSubcoreMesh(
        core_axis_name="c", subcore_axis_name="s", num_cores=NC, num_subcores=NS
    )

    @functools.partial(
        pl.kernel,
        out_type=jax.ShapeDtypeStruct((n_rows, D), jnp.float32),
        mesh=mesh,
        scratch_types=[
            pltpu.VMEM((CH,), jnp.int32),
            pltpu.VMEM((CH, D), jnp.float32),
            pltpu.VMEM((2, D), jnp.float32),
            pltpu.SemaphoreType.DMA,
        ],
        compiler_params=pltpu.CompilerParams(use_tc_tiling_on_sc=False),
    )
    def k(x_hbm, table_hbm, gb_hbm, out_hbm, idx_v, rows_v, gb_v, sem):
        wid = lax.axis_index("s") * NC + lax.axis_index("c")
        base = wid * rows_per_worker
        pltpu.sync_copy(gb_hbm, gb_v)
        perms = _butterfly_perms()

        def chunk_body(j, carry):
            off = base + j * CH
            pltpu.sync_copy(x_hbm.at[pl.ds(off, CH)], idx_v)
            pltpu.async_copy(table_hbm.at[idx_v], rows_v, sem).wait()

            def row_body(r, c2):
                v0 = rows_v[r, pl.ds(0, L)]
                v1 = rows_v[r, pl.ds(L, L)]
                v2 = rows_v[r, pl.ds(2 * L, L)]
                v3 = rows_v[r, pl.ds(3 * L, L)]
                s = (v0 + v1) + (v2 + v3)
                q = (v0 * v0 + v1 * v1) + (v2 * v2 + v3 * v3)
                ssum = _lane_sum(s, perms)
                qsum = _lane_sum(q, perms)
                mean = ssum * jnp.float32(1.0 / D)
                var = qsum * jnp.float32(1.0 / D) - mean * mean
                kk = _rsqrt_vec(var + jnp.float32(EPS))
                for i, v in enumerate((v0, v1, v2, v3)):
                    g = gb_v[0, pl.ds(i * L, L)]
                    b = gb_v[1, pl.ds(i * L, L)]
                    rows_v[r, pl.ds(i * L, L)] = (v - mean) * kk * g + b
                return c2

            lax.fori_loop(0, CH, row_body, 0, unroll=2)
            pltpu.sync_copy(rows_v, out_hbm.at[pl.ds(off, CH)])
            return carry

        lax.fori_loop(0, n_chunks, chunk_body, 0)

    return k


def kernel(x, table, gamma, beta):
    b, h = x.shape
    n_rows = b * h
    xf = x.reshape((n_rows,)).astype(jnp.int32)
    gb = jnp.stack([gamma, beta]).astype(jnp.float32)
    out = _make_kernel(n_rows)(xf, table, gb)
    return out.reshape((b, h, D))
